# baseline (speedup 1.0000x reference)
import jax
import jax.numpy as jnp
from jax.experimental import pallas as pl
from jax.experimental.pallas import tpu as pltpu

_N = 10000
_F = 128
_BM = 80
_NB = _N // _BM
_S = 29
_NS = _NB - _S
_NBUF = 2


def _fetch(a_ref, buf_ref, sem_ref, block, slot):
    return pltpu.make_async_copy(
        a_ref.at[pl.ds(block * _BM, _BM), :], buf_ref.at[slot],
        sem_ref.at[slot])


def _put(outv_ref, out_ref, sem_ref, block, slot):
    return pltpu.make_async_copy(
        outv_ref.at[slot], out_ref.at[pl.ds(block * _BM, _BM), :],
        sem_ref.at[slot])


_CAD = 3


def _p2_block(t):
    m = t // _CAD
    cached = (t % _CAD == 0) & (t >= _CAD) & (t <= _CAD * _S)
    j = jnp.where(cached, _NS + m - 1, t - jnp.minimum(m, _S))
    return j, cached, m


def _gcn_kernel(x_ref, a_ref, w1_ref, b1_ref, w2_ref, b2_ref, out_ref,
                buf_ref, y1_ref, y2b_ref, cache_ref, outv_ref,
                in_sem, out_sem):
    for b in range(_NBUF):
        _fetch(a_ref, buf_ref, in_sem, b, b).start()

    y1 = jax.lax.dot_general(
        x_ref[...], w1_ref[...], (((1,), (1,)), ((), ())),
        preferred_element_type=jnp.float32)
    y1_ref[...] = y1 + b1_ref[...]

    def pass1(k, carry):
        sl = jax.lax.rem(k, _NBUF)
        _fetch(a_ref, buf_ref, in_sem, k, sl).wait()
        h = jnp.dot(buf_ref[sl], y1_ref[...],
                    preferred_element_type=jnp.float32)
        h = jnp.maximum(h, 0.0)
        y2 = jax.lax.dot_general(h, w2_ref[...], (((1,), (1,)), ((), ())),
                                 preferred_element_type=jnp.float32)
        y2 = y2 + b2_ref[...]
        y2b_ref[k] = y2.astype(jnp.bfloat16)

        @pl.when(k >= _NB - _S)
        def _store_cache():
            cache_ref[k - (_NB - _S)] = buf_ref[sl].astype(jnp.bfloat16)

        g = k + _NBUF
        nxt = jnp.where(g < _NB, g, g - _NB)
        _fetch(a_ref, buf_ref, in_sem, nxt, sl).start()
        return carry

    jax.lax.fori_loop(0, _NB, pass1, 0)

    def pass2(t, carry):
        j, cached, m = _p2_block(t)
        ov = jax.lax.rem(t, 2)

        @pl.when(t >= 2)
        def _wait_out():
            jprev, _, _ = _p2_block(t - 2)
            _put(outv_ref, out_ref, out_sem, jprev, ov).wait()

        @pl.when(cached)
        def _cached():
            outv_ref[ov] = jnp.dot(cache_ref[m - 1],
                                   y2b_ref[...].reshape(_N, _F),
                                   preferred_element_type=jnp.float32)

        @pl.when(jnp.logical_not(cached))
        def _streamed():
            sl = jax.lax.rem(j, _NBUF)
            _fetch(a_ref, buf_ref, in_sem, j, sl).wait()
            outv_ref[ov] = jax.lax.dot_general(
                buf_ref[sl], y2b_ref[...].reshape(_N, _F),
                (((1,), (0,)), ((), ())),
                preferred_element_type=jnp.float32)

            @pl.when(j + _NBUF < _NS)
            def _issue():
                _fetch(a_ref, buf_ref, in_sem, j + _NBUF, sl).start()

        _put(outv_ref, out_ref, out_sem, j, ov).start()
        return carry

    jax.lax.fori_loop(0, _NB, pass2, 0)

    for t in (_NB - 2, _NB - 1):
        m = t // _CAD
        cached = (t % _CAD == 0) and (t >= _CAD) and (t <= _CAD * _S)
        j = (_NS + m - 1) if cached else (t - min(m, _S))
        _put(outv_ref, out_ref, out_sem, j, t % 2).wait()


def kernel(X, A, W1, b1, W2, b2):
    b1r = b1.reshape(1, _F)
    b2r = b2.reshape(1, _F)

    out = pl.pallas_call(
        _gcn_kernel,
        in_specs=[
            pl.BlockSpec(memory_space=pltpu.MemorySpace.VMEM),
            pl.BlockSpec(memory_space=pl.ANY),
            pl.BlockSpec(memory_space=pltpu.MemorySpace.VMEM),
            pl.BlockSpec(memory_space=pltpu.MemorySpace.VMEM),
            pl.BlockSpec(memory_space=pltpu.MemorySpace.VMEM),
            pl.BlockSpec(memory_space=pltpu.MemorySpace.VMEM),
        ],
        out_specs=pl.BlockSpec(memory_space=pl.ANY),
        out_shape=jax.ShapeDtypeStruct((_N, _F), jnp.float32),
        scratch_shapes=[
            pltpu.VMEM((_NBUF, _BM, _N), jnp.float32),
            pltpu.VMEM((_N, _F), jnp.float32),
            pltpu.VMEM((_NB, _BM, _F), jnp.bfloat16),
            pltpu.VMEM((_S, _BM, _N), jnp.bfloat16),
            pltpu.VMEM((2, _BM, _F), jnp.float32),
            pltpu.SemaphoreType.DMA((_NBUF,)),
            pltpu.SemaphoreType.DMA((2,)),
        ],
        compiler_params=pltpu.CompilerParams(
            vmem_limit_bytes=64 * 1024 * 1024),
    )(X, A, W1, b1r, W2, b2r)
    return out


# speedup vs baseline: 1.0056x; 1.0056x over previous
import jax
import jax.numpy as jnp
from jax.experimental import pallas as pl
from jax.experimental.pallas import tpu as pltpu

_N = 10000
_F = 128
_BM = 80
_NB = _N // _BM
_S = 29
_NS = _NB - _S
_NBUF = 2


def _fetch(a_ref, buf_ref, sem_ref, block, slot):
    return pltpu.make_async_copy(
        a_ref.at[pl.ds(block * _BM, _BM), :], buf_ref.at[slot],
        sem_ref.at[slot])


def _put(outv_ref, out_ref, sem_ref, block, slot):
    return pltpu.make_async_copy(
        outv_ref.at[slot], out_ref.at[pl.ds(block * _BM, _BM), :],
        sem_ref.at[slot])


_CAD = 3


def _p2_block(t):
    m = t // _CAD
    cached = (t % _CAD == 0) & (t >= _CAD) & (t <= _CAD * _S)
    j = jnp.where(cached, _NS + m - 1, t - jnp.minimum(m, _S))
    return j, cached, m


def _gcn_kernel(x_ref, a_ref, w1_ref, b1_ref, w2_ref, b2_ref, out_ref,
                buf_ref, y1_ref, y2b_ref, cache_ref, outv_ref,
                in_sem, out_sem):
    for b in range(_NBUF):
        _fetch(a_ref, buf_ref, in_sem, b, b).start()

    y1 = jax.lax.dot_general(
        x_ref[...], w1_ref[...], (((1,), (1,)), ((), ())),
        preferred_element_type=jnp.float32)
    y1_ref[...] = y1 + b1_ref[...]

    def pass1(k, carry):
        sl = jax.lax.rem(k, _NBUF)
        _fetch(a_ref, buf_ref, in_sem, k, sl).wait()
        h = jnp.dot(buf_ref[sl], y1_ref[...],
                    preferred_element_type=jnp.float32)
        h = jnp.maximum(h, 0.0)
        y2 = jax.lax.dot_general(h, w2_ref[...], (((1,), (1,)), ((), ())),
                                 preferred_element_type=jnp.float32)
        y2 = y2 + b2_ref[...]
        y2b_ref[k] = y2.astype(jnp.bfloat16)

        @pl.when(k >= _NB - _S)
        def _store_cache():
            cache_ref[k - (_NB - _S)] = buf_ref[sl].astype(jnp.bfloat16)

        g = k + _NBUF
        nxt = jnp.where(g < _NB, g, g - _NB)
        _fetch(a_ref, buf_ref, in_sem, nxt, sl).start()
        return carry

    jax.lax.fori_loop(0, _NB, pass1, 0)

    def pass2(t, carry):
        j, cached, m = _p2_block(t)
        ov = jax.lax.rem(t, 2)

        @pl.when(t >= 2)
        def _wait_out():
            jprev, _, _ = _p2_block(t - 2)
            _put(outv_ref, out_ref, out_sem, jprev, ov).wait()

        @pl.when(cached)
        def _cached():
            outv_ref[ov] = jnp.dot(cache_ref[m - 1],
                                   y2b_ref[...].reshape(_N, _F),
                                   preferred_element_type=jnp.float32)

        @pl.when(jnp.logical_not(cached))
        def _streamed():
            sl = jax.lax.rem(j + _NB, _NBUF)
            _fetch(a_ref, buf_ref, in_sem, j, sl).wait()
            outv_ref[ov] = jax.lax.dot_general(
                buf_ref[sl], y2b_ref[...].reshape(_N, _F),
                (((1,), (0,)), ((), ())),
                preferred_element_type=jnp.float32)

            @pl.when(j + _NBUF < _NS)
            def _issue():
                _fetch(a_ref, buf_ref, in_sem, j + _NBUF, sl).start()

        _put(outv_ref, out_ref, out_sem, j, ov).start()
        return carry

    jax.lax.fori_loop(0, _NB, pass2, 0)

    for t in (_NB - 2, _NB - 1):
        m = t // _CAD
        cached = (t % _CAD == 0) and (t >= _CAD) and (t <= _CAD * _S)
        j = (_NS + m - 1) if cached else (t - min(m, _S))
        _put(outv_ref, out_ref, out_sem, j, t % 2).wait()


def kernel(X, A, W1, b1, W2, b2):
    b1r = b1.reshape(1, _F)
    b2r = b2.reshape(1, _F)

    out = pl.pallas_call(
        _gcn_kernel,
        in_specs=[
            pl.BlockSpec(memory_space=pltpu.MemorySpace.VMEM),
            pl.BlockSpec(memory_space=pl.ANY),
            pl.BlockSpec(memory_space=pltpu.MemorySpace.VMEM),
            pl.BlockSpec(memory_space=pltpu.MemorySpace.VMEM),
            pl.BlockSpec(memory_space=pltpu.MemorySpace.VMEM),
            pl.BlockSpec(memory_space=pltpu.MemorySpace.VMEM),
        ],
        out_specs=pl.BlockSpec(memory_space=pl.ANY),
        out_shape=jax.ShapeDtypeStruct((_N, _F), jnp.float32),
        scratch_shapes=[
            pltpu.VMEM((_NBUF, _BM, _N), jnp.float32),
            pltpu.VMEM((_N, _F), jnp.float32),
            pltpu.VMEM((_NB, _BM, _F), jnp.bfloat16),
            pltpu.VMEM((_S, _BM, _N), jnp.bfloat16),
            pltpu.VMEM((2, _BM, _F), jnp.float32),
            pltpu.SemaphoreType.DMA((_NBUF,)),
            pltpu.SemaphoreType.DMA((2,)),
        ],
        compiler_params=pltpu.CompilerParams(
            vmem_limit_bytes=64 * 1024 * 1024),
    )(X, A, W1, b1r, W2, b2r)
    return out


# speedup vs baseline: 1.3416x; 1.3342x over previous
import jax
import jax.numpy as jnp
from jax.experimental import pallas as pl
from jax.experimental.pallas import tpu as pltpu

_N = 10000
_F = 128
_BM = 200
_NB = _N // _BM
_S = 9
_NS = _NB - _S
_NBUF = 2


def _fetch(a_ref, buf_ref, sem_ref, block, slot):
    return pltpu.make_async_copy(
        a_ref.at[pl.ds(block * _BM, _BM), :], buf_ref.at[slot],
        sem_ref.at[slot])


def _put(outv_ref, out_ref, sem_ref, block, slot):
    return pltpu.make_async_copy(
        outv_ref.at[slot], out_ref.at[pl.ds(block * _BM, _BM), :],
        sem_ref.at[slot])


def _p2_block(t):
    m = t // 5
    cached = (t % 5 == 0) & (t >= 5) & (t <= 5 * _S)
    j = jnp.where(cached, _NS + m - 1, t - jnp.minimum(m, _S))
    return j, cached, m


def _gcn_kernel(x_ref, a_ref, w1_ref, b1_ref, w2_ref, b2_ref, out_ref,
                buf_ref, y1_ref, y2b_ref, cache_ref, outv_ref,
                in_sem, out_sem):
    for b in range(_NBUF):
        _fetch(a_ref, buf_ref, in_sem, b, b).start()

    y1 = jax.lax.dot_general(
        x_ref[...], w1_ref[...], (((1,), (1,)), ((), ())),
        preferred_element_type=jnp.float32)
    y1_ref[...] = y1 + b1_ref[...]

    def pass1(k, carry):
        sl = jax.lax.rem(k, _NBUF)
        _fetch(a_ref, buf_ref, in_sem, k, sl).wait()
        h = jnp.dot(buf_ref[sl], y1_ref[...],
                    preferred_element_type=jnp.float32)
        h = jnp.maximum(h, 0.0)
        y2 = jax.lax.dot_general(h, w2_ref[...], (((1,), (1,)), ((), ())),
                                 preferred_element_type=jnp.float32)
        y2 = y2 + b2_ref[...]
        y2b_ref[k] = y2.astype(jnp.bfloat16)

        @pl.when(k >= _NB - _S)
        def _store_cache():
            cache_ref[k - (_NB - _S)] = buf_ref[sl].astype(jnp.bfloat16)

        g = k + _NBUF
        nxt = jnp.where(g < _NB, g, g - _NB)
        _fetch(a_ref, buf_ref, in_sem, nxt, sl).start()
        return carry

    jax.lax.fori_loop(0, _NB, pass1, 0)

    def pass2(t, carry):
        j, cached, m = _p2_block(t)
        ov = jax.lax.rem(t, 2)

        @pl.when(t >= 2)
        def _wait_out():
            jprev, _, _ = _p2_block(t - 2)
            _put(outv_ref, out_ref, out_sem, jprev, ov).wait()

        @pl.when(cached)
        def _cached():
            outv_ref[ov] = jnp.dot(cache_ref[m - 1],
                                   y2b_ref[...].reshape(_N, _F),
                                   preferred_element_type=jnp.float32)

        @pl.when(jnp.logical_not(cached))
        def _streamed():
            sl = jax.lax.rem(j, _NBUF)
            _fetch(a_ref, buf_ref, in_sem, j, sl).wait()
            outv_ref[ov] = jax.lax.dot_general(
                buf_ref[sl], y2b_ref[...].reshape(_N, _F),
                (((1,), (0,)), ((), ())),
                preferred_element_type=jnp.float32)

            @pl.when(j + _NBUF < _NS)
            def _issue():
                _fetch(a_ref, buf_ref, in_sem, j + _NBUF, sl).start()

        _put(outv_ref, out_ref, out_sem, j, ov).start()
        return carry

    jax.lax.fori_loop(0, _NB, pass2, 0)

    for t in (_NB - 2, _NB - 1):
        m = t // 5
        cached = (t % 5 == 0) and (t >= 5) and (t <= 5 * _S)
        j = (_NS + m - 1) if cached else (t - min(m, _S))
        _put(outv_ref, out_ref, out_sem, j, t % 2).wait()


def kernel(X, A, W1, b1, W2, b2):
    b1r = b1.reshape(1, _F)
    b2r = b2.reshape(1, _F)

    out = pl.pallas_call(
        _gcn_kernel,
        in_specs=[
            pl.BlockSpec(memory_space=pltpu.MemorySpace.VMEM),
            pl.BlockSpec(memory_space=pl.ANY),
            pl.BlockSpec(memory_space=pltpu.MemorySpace.VMEM),
            pl.BlockSpec(memory_space=pltpu.MemorySpace.VMEM),
            pl.BlockSpec(memory_space=pltpu.MemorySpace.VMEM),
            pl.BlockSpec(memory_space=pltpu.MemorySpace.VMEM),
        ],
        out_specs=pl.BlockSpec(memory_space=pl.ANY),
        out_shape=jax.ShapeDtypeStruct((_N, _F), jnp.float32),
        scratch_shapes=[
            pltpu.VMEM((_NBUF, _BM, _N), jnp.float32),
            pltpu.VMEM((_N, _F), jnp.float32),
            pltpu.VMEM((_NB, _BM, _F), jnp.bfloat16),
            pltpu.VMEM((_S, _BM, _N), jnp.bfloat16),
            pltpu.VMEM((2, _BM, _F), jnp.float32),
            pltpu.SemaphoreType.DMA((_NBUF,)),
            pltpu.SemaphoreType.DMA((2,)),
        ],
        compiler_params=pltpu.CompilerParams(
            vmem_limit_bytes=64 * 1024 * 1024),
    )(X, A, W1, b1r, W2, b2r)
    return out
